# baseline (speedup 1.0000x reference)
import functools
import math

import jax
import jax.numpy as jnp
from jax import lax
from jax.experimental import pallas as pl
from jax.experimental.pallas import tpu as pltpu
from jax.experimental.pallas import tpu_sc as plsc

_C = 1.0
_VOCAB = 1000000
_DIM = 64
_BATCH = 4096
_SEQ = 50

_NC = 2
_NS = 16
_NW = _NC * _NS

_TOKENS = _BATCH * _SEQ
_PER_W = _TOKENS // _NW
_CHUNK = 128
_NCHUNK = _PER_W // _CHUNK

_TC_ROWS = 1600
_TC_GRID = _TOKENS // _TC_ROWS


def _sc_gather(idx3, table):
  mesh = plsc.VectorSubcoreMesh(
      core_axis_name="c", subcore_axis_name="s",
      num_cores=_NC, num_subcores=_NS)

  @functools.partial(
      pl.kernel,
      mesh=mesh,
      compiler_params=pltpu.CompilerParams(use_tc_tiling_on_sc=False),
      out_type=jax.ShapeDtypeStruct((_TOKENS, _DIM), jnp.float32),
      scratch_types=[
          pltpu.VMEM((_CHUNK,), jnp.int32),
          pltpu.VMEM((_CHUNK, _DIM), jnp.float32),
          pltpu.SemaphoreType.DMA,
      ],
  )
  def k(idx_hbm, table_hbm, out_hbm, idx_v, rows_v, gsem):
    wid = lax.axis_index("s") * _NC + lax.axis_index("c")
    base = wid * _PER_W

    def body(j, _):
      pltpu.sync_copy(idx_hbm.at[wid, j], idx_v)
      pltpu.async_copy(table_hbm.at[idx_v], rows_v, gsem).wait()
      pltpu.sync_copy(rows_v, out_hbm.at[pl.ds(base + j * _CHUNK, _CHUNK)])
      return 0

    lax.fori_loop(0, _NCHUNK, body, 0)

  return k(idx3, table)


def _tc_body(x_ref, pe_ref, wt_ref, b_ref, sc_ref, out_ref):
  x = x_ref[...]
  y = x + pe_ref[...]
  sq = y * y
  neg_inner = 2.0 * sq[:, 0:1] - jnp.sum(sq, axis=1, keepdims=True)
  inv = lax.rsqrt(jnp.maximum(neg_inner, 1e-7))
  y = y * inv
  h = jnp.dot(y, wt_ref[...], preferred_element_type=jnp.float32) + b_ref[...]
  h0 = h[:, 0:1]
  time = jax.nn.sigmoid(h0) * sc_ref[0, 1] + 1.1
  ssq = jnp.sum(h * h, axis=1, keepdims=True) - h0 * h0
  s = (time * time - 1.0 / _C) / jnp.maximum(ssq, 1e-8)
  scaled = h[:, 0:63] * jnp.sqrt(s)
  colid = lax.broadcasted_iota(jnp.int32, (_TC_ROWS, 63), 1)
  out_ref[...] = jnp.where(colid == 0, time, scaled)


def _tc_transform(gathered, pe_block, wt_pad, b_pad, scalars, interpret=False):
  return pl.pallas_call(
      _tc_body,
      grid=(_TC_GRID,),
      in_specs=[
          pl.BlockSpec((_TC_ROWS, _DIM), lambda i: (i, 0)),
          pl.BlockSpec((_TC_ROWS, _DIM), lambda i: (0, 0)),
          pl.BlockSpec((_DIM, _DIM), lambda i: (0, 0)),
          pl.BlockSpec((1, _DIM), lambda i: (0, 0)),
          pl.BlockSpec((1, 2), lambda i: (0, 0)),
      ],
      out_specs=pl.BlockSpec((_TC_ROWS, 63), lambda i: (i, 0)),
      out_shape=jax.ShapeDtypeStruct((_TOKENS, 63), jnp.float32),
      interpret=interpret,
  )(gathered, pe_block, wt_pad, b_pad, scalars)


def kernel(source, embedding, pos_enc, add_scale, W, b, point_scale):
  idx = source.reshape(-1).astype(jnp.int32)
  idx3 = idx.reshape(_NW, _NCHUNK, _CHUNK)

  gathered = _sc_gather(idx3, embedding)

  pe = (add_scale * pos_enc[:_SEQ, 0, :]).astype(jnp.float32)
  pe_block = jnp.tile(pe, (_TC_ROWS // _SEQ, 1))
  wt_pad = jnp.zeros((_DIM, _DIM), jnp.float32).at[:, :63].set(W.T)
  b_pad = jnp.zeros((1, _DIM), jnp.float32).at[0, :63].set(b)
  scalars = jnp.stack([add_scale, jnp.exp(point_scale)]).reshape(1, 2)

  out = _tc_transform(gathered, pe_block, wt_pad, b_pad, scalars)
  return out.reshape(_BATCH, _SEQ, 63)


# speedup vs baseline: 1.0561x; 1.0561x over previous
import functools
import math

import jax
import jax.numpy as jnp
from jax import lax
from jax.experimental import pallas as pl
from jax.experimental.pallas import tpu as pltpu
from jax.experimental.pallas import tpu_sc as plsc

_C = 1.0
_VOCAB = 1000000
_DIM = 64
_BATCH = 4096
_SEQ = 50

_NC = 2
_NS = 16
_NW = _NC * _NS

_TOKENS = _BATCH * _SEQ
_PER_W = _TOKENS // _NW
_CHUNK = 128
_NCHUNK = _PER_W // _CHUNK
_NSLOT = 5

_TC_ROWS = 1600
_TC_GRID = _TOKENS // _TC_ROWS


def _sc_gather(idx3, table):
  mesh = plsc.VectorSubcoreMesh(
      core_axis_name="c", subcore_axis_name="s",
      num_cores=_NC, num_subcores=_NS)

  @functools.partial(
      pl.kernel,
      mesh=mesh,
      compiler_params=pltpu.CompilerParams(use_tc_tiling_on_sc=False),
      out_type=jax.ShapeDtypeStruct((_TOKENS, _DIM), jnp.float32),
      scratch_types=[
          pltpu.VMEM((_NCHUNK, _CHUNK), jnp.int32),
          pltpu.VMEM((_NSLOT, _CHUNK, _DIM), jnp.float32),
          pltpu.SemaphoreType.DMA((_NSLOT,)),
          pltpu.SemaphoreType.DMA((_NSLOT,)),
      ],
  )
  def k(idx_hbm, table_hbm, out_hbm, idx_v, rows_v, gsem, wsem):
    wid = lax.axis_index("s") * _NC + lax.axis_index("c")
    base = wid * _PER_W
    pltpu.sync_copy(idx_hbm.at[wid], idx_v)

    def gather(j, s):
      pltpu.make_async_copy(
          table_hbm.at[idx_v.at[j]], rows_v.at[s], gsem.at[s]).start()

    def writeback(j, s):
      return pltpu.make_async_copy(
          rows_v.at[s], out_hbm.at[pl.ds(base + j * _CHUNK, _CHUNK)],
          wsem.at[s])

    for s in range(_NSLOT):
      gather(s, s)

    def body(jj, _):
      for s in range(_NSLOT):
        j = jj * _NSLOT + s
        pltpu.make_async_copy(
            table_hbm.at[idx_v.at[j]], rows_v.at[s], gsem.at[s]).wait()
        writeback(j, s).start()

        @pl.when(jj < _NCHUNK // _NSLOT - 1)
        def _():
          writeback(j, s).wait()
          gather(j + _NSLOT, s)

      return 0

    lax.fori_loop(0, _NCHUNK // _NSLOT, body, 0)

    for s in range(_NSLOT):
      writeback(_NCHUNK - _NSLOT + s, s).wait()

  return k(idx3, table)


def _tc_body(x_ref, pe_ref, wt_ref, b_ref, sc_ref, out_ref):
  x = x_ref[...]
  y = x + pe_ref[...]
  sq = y * y
  neg_inner = 2.0 * sq[:, 0:1] - jnp.sum(sq, axis=1, keepdims=True)
  inv = lax.rsqrt(jnp.maximum(neg_inner, 1e-7))
  y = y * inv
  h = jnp.dot(y, wt_ref[...], preferred_element_type=jnp.float32) + b_ref[...]
  h0 = h[:, 0:1]
  time = jax.nn.sigmoid(h0) * sc_ref[0, 1] + 1.1
  ssq = jnp.sum(h * h, axis=1, keepdims=True) - h0 * h0
  s = (time * time - 1.0 / _C) / jnp.maximum(ssq, 1e-8)
  scaled = h[:, 0:63] * jnp.sqrt(s)
  colid = lax.broadcasted_iota(jnp.int32, (_TC_ROWS, 63), 1)
  out_ref[...] = jnp.where(colid == 0, time, scaled)


def _tc_transform(gathered, pe_block, wt_pad, b_pad, scalars, interpret=False):
  return pl.pallas_call(
      _tc_body,
      grid=(_TC_GRID,),
      in_specs=[
          pl.BlockSpec((_TC_ROWS, _DIM), lambda i: (i, 0)),
          pl.BlockSpec((_TC_ROWS, _DIM), lambda i: (0, 0)),
          pl.BlockSpec((_DIM, _DIM), lambda i: (0, 0)),
          pl.BlockSpec((1, _DIM), lambda i: (0, 0)),
          pl.BlockSpec((1, 2), lambda i: (0, 0)),
      ],
      out_specs=pl.BlockSpec((_TC_ROWS, 63), lambda i: (i, 0)),
      out_shape=jax.ShapeDtypeStruct((_TOKENS, 63), jnp.float32),
      interpret=interpret,
  )(gathered, pe_block, wt_pad, b_pad, scalars)


def kernel(source, embedding, pos_enc, add_scale, W, b, point_scale):
  idx = source.reshape(-1).astype(jnp.int32)
  idx3 = idx.reshape(_NW, _NCHUNK, _CHUNK)

  gathered = _sc_gather(idx3, embedding)

  pe = (add_scale * pos_enc[:_SEQ, 0, :]).astype(jnp.float32)
  pe_block = jnp.tile(pe, (_TC_ROWS // _SEQ, 1))
  wt_pad = jnp.zeros((_DIM, _DIM), jnp.float32).at[:, :63].set(W.T)
  b_pad = jnp.zeros((1, _DIM), jnp.float32).at[0, :63].set(b)
  scalars = jnp.stack([add_scale, jnp.exp(point_scale)]).reshape(1, 2)

  out = _tc_transform(gathered, pe_block, wt_pad, b_pad, scalars)
  return out.reshape(_BATCH, _SEQ, 63)


# speedup vs baseline: 1.3513x; 1.2796x over previous
import functools
import math

import jax
import jax.numpy as jnp
from jax import lax
from jax.experimental import pallas as pl
from jax.experimental.pallas import tpu as pltpu
from jax.experimental.pallas import tpu_sc as plsc

_C = 1.0
_VOCAB = 1000000
_DIM = 64
_BATCH = 4096
_SEQ = 50

_NC = 2
_NS = 16
_NW = _NC * _NS

_TOKENS = _BATCH * _SEQ
_PER_W = _TOKENS // _NW
_CHUNK = 128
_NCHUNK = _PER_W // _CHUNK
_NSLOT = 5

_BBLK = 2048
_NB = _BATCH // _BBLK


def _sc_gather(idx3, table):
  mesh = plsc.VectorSubcoreMesh(
      core_axis_name="c", subcore_axis_name="s",
      num_cores=_NC, num_subcores=_NS)

  @functools.partial(
      pl.kernel,
      mesh=mesh,
      compiler_params=pltpu.CompilerParams(use_tc_tiling_on_sc=False),
      out_type=jax.ShapeDtypeStruct((_TOKENS, _DIM), jnp.float32),
      scratch_types=[
          pltpu.VMEM((_NCHUNK, _CHUNK), jnp.int32),
          pltpu.VMEM((_NSLOT, _CHUNK, _DIM), jnp.float32),
          pltpu.SemaphoreType.DMA((_NSLOT,)),
          pltpu.SemaphoreType.DMA((_NSLOT,)),
      ],
  )
  def k(idx_hbm, table_hbm, out_hbm, idx_v, rows_v, gsem, wsem):
    wid = lax.axis_index("s") * _NC + lax.axis_index("c")
    base = wid * _PER_W
    pltpu.sync_copy(idx_hbm.at[wid], idx_v)

    def gather(j, s):
      pltpu.make_async_copy(
          table_hbm.at[idx_v.at[j]], rows_v.at[s], gsem.at[s]).start()

    def writeback(j, s):
      return pltpu.make_async_copy(
          rows_v.at[s], out_hbm.at[pl.ds(base + j * _CHUNK, _CHUNK)],
          wsem.at[s])

    for s in range(_NSLOT):
      gather(s, s)

    def body(jj, _):
      for s in range(_NSLOT):
        j = jj * _NSLOT + s
        pltpu.make_async_copy(
            table_hbm.at[idx_v.at[j]], rows_v.at[s], gsem.at[s]).wait()
        writeback(j, s).start()

        @pl.when(jj < _NCHUNK // _NSLOT - 1)
        def _():
          writeback(j, s).wait()
          gather(j + _NSLOT, s)

      return 0

    lax.fori_loop(0, _NCHUNK // _NSLOT, body, 0)

    for s in range(_NSLOT):
      writeback(_NCHUNK - _NSLOT + s, s).wait()

  return k(idx3, table)


def _tc_body(x_ref, pe_ref, w_ref, b_ref, sc_ref, out_ref):
  x = x_ref[0]
  ident = (lax.broadcasted_iota(jnp.int32, (_DIM, _DIM), 0) ==
           lax.broadcasted_iota(jnp.int32, (_DIM, _DIM), 1)).astype(jnp.float32)
  xt = lax.dot_general(ident, x, (((1,), (1,)), ((), ())),
                       preferred_element_type=jnp.float32)
  pe = jnp.reshape(pe_ref[...], (_DIM, 1))
  y = xt + pe
  sq = y * y
  colsum = jnp.sum(sq, axis=0, keepdims=True)
  y0 = y[0:1, :]
  inv = lax.rsqrt(jnp.maximum(2.0 * y0 * y0 - colsum, 1e-7))
  yn = y * inv
  h = lax.dot_general(w_ref[...], yn, (((1,), (0,)), ((), ())),
                      preferred_element_type=jnp.float32) + b_ref[...]
  time = jax.nn.sigmoid(h[0:1, :]) * sc_ref[0, 0] + 1.1
  ssq = jnp.sum(h * h, axis=0, keepdims=True) - h[0:1, :] * h[0:1, :]
  s = (time * time - 1.0 / _C) / jnp.maximum(ssq, 1e-8)
  scaled = h * jnp.sqrt(s)
  rowid = lax.broadcasted_iota(jnp.int32, (_DIM, _BBLK), 0)
  outv = jnp.where(rowid == 0, time, scaled)
  out_ref[...] = jnp.reshape(outv[0:63, :], (1, 63, _BBLK))


def _tc_transform(gathered3, pe_t, w_pad, b_col, scalars, interpret=False):
  return pl.pallas_call(
      _tc_body,
      grid=(_SEQ, _NB),
      in_specs=[
          pl.BlockSpec((1, _BBLK, _DIM), lambda i, j: (i, j, 0)),
          pl.BlockSpec((1, _DIM, 1), lambda i, j: (i, 0, 0)),
          pl.BlockSpec((_DIM, _DIM), lambda i, j: (0, 0)),
          pl.BlockSpec((_DIM, 1), lambda i, j: (0, 0)),
          pl.BlockSpec((1, 1), lambda i, j: (0, 0)),
      ],
      out_specs=pl.BlockSpec((1, 63, _BBLK), lambda i, j: (i, 0, j)),
      out_shape=jax.ShapeDtypeStruct((_SEQ, 63, _BATCH), jnp.float32),
      interpret=interpret,
  )(gathered3, pe_t, w_pad, b_col, scalars)


def kernel(source, embedding, pos_enc, add_scale, W, b, point_scale):
  idx = jnp.transpose(source).reshape(-1).astype(jnp.int32)
  idx3 = idx.reshape(_NW, _NCHUNK, _CHUNK)

  gathered = _sc_gather(idx3, embedding)
  gathered3 = gathered.reshape(_SEQ, _BATCH, _DIM)

  pe_t = (add_scale * pos_enc[:_SEQ, 0, :]).astype(jnp.float32)[:, :, None]
  w_pad = jnp.zeros((_DIM, _DIM), jnp.float32).at[:63, :].set(W)
  b_col = jnp.zeros((_DIM, 1), jnp.float32).at[:63, 0].set(b)
  scalars = jnp.exp(point_scale).reshape(1, 1)

  out = _tc_transform(gathered3, pe_t, w_pad, b_col, scalars)
  return jnp.transpose(out, (2, 0, 1))


# speedup vs baseline: 1.4215x; 1.0519x over previous
import functools
import math

import jax
import jax.numpy as jnp
from jax import lax
from jax.experimental import pallas as pl
from jax.experimental.pallas import tpu as pltpu
from jax.experimental.pallas import tpu_sc as plsc

_C = 1.0
_VOCAB = 1000000
_DIM = 64
_BATCH = 4096
_SEQ = 50

_NC = 2
_NS = 16
_NW = _NC * _NS

_TOKENS = _BATCH * _SEQ
_PER_W = _TOKENS // _NW
_CHUNK = 128
_NCHUNK = _PER_W // _CHUNK
_NSLOT = 5

_BBLK = 2048
_NB = _BATCH // _BBLK


_VROWS = _VOCAB // 2


def _sc_gather(idx3, table):
  mesh = plsc.VectorSubcoreMesh(
      core_axis_name="c", subcore_axis_name="s",
      num_cores=_NC, num_subcores=_NS)

  @functools.partial(
      pl.kernel,
      mesh=mesh,
      compiler_params=pltpu.CompilerParams(use_tc_tiling_on_sc=False),
      out_type=jax.ShapeDtypeStruct((_TOKENS, 2 * _DIM), jnp.float32),
      scratch_types=[
          pltpu.VMEM((_NCHUNK, _CHUNK), jnp.int32),
          pltpu.VMEM((_NSLOT, _CHUNK, 2 * _DIM), jnp.float32),
          pltpu.SemaphoreType.DMA((_NSLOT,)),
          pltpu.SemaphoreType.DMA((_NSLOT,)),
      ],
  )
  def k(idx_hbm, table_hbm, out_hbm, idx_v, rows_v, gsem, wsem):
    wid = lax.axis_index("s") * _NC + lax.axis_index("c")
    base = wid * _PER_W
    pltpu.sync_copy(idx_hbm.at[wid], idx_v)

    def gather(j, s):
      pltpu.make_async_copy(
          table_hbm.at[idx_v.at[j]], rows_v.at[s], gsem.at[s]).start()

    def writeback(j, s):
      return pltpu.make_async_copy(
          rows_v.at[s], out_hbm.at[pl.ds(base + j * _CHUNK, _CHUNK)],
          wsem.at[s])

    for s in range(_NSLOT):
      gather(s, s)

    def body(jj, _):
      for s in range(_NSLOT):
        j = jj * _NSLOT + s
        pltpu.make_async_copy(
            table_hbm.at[idx_v.at[j]], rows_v.at[s], gsem.at[s]).wait()
        writeback(j, s).start()

        @pl.when(jj < _NCHUNK // _NSLOT - 1)
        def _():
          writeback(j, s).wait()
          gather(j + _NSLOT, s)

      return 0

    lax.fori_loop(0, _NCHUNK // _NSLOT, body, 0)

    for s in range(_NSLOT):
      writeback(_NCHUNK - _NSLOT + s, s).wait()

  return k(idx3, table)


def _tc_body(x_ref, par_ref, pe_ref, w_ref, b_ref, sc_ref, out_ref):
  x = x_ref[...]
  ident = (lax.broadcasted_iota(jnp.int32, (128, 128), 0) ==
           lax.broadcasted_iota(jnp.int32, (128, 128), 1)).astype(jnp.float32)
  xt = lax.dot_general(ident, x, (((1,), (1,)), ((), ())),
                       preferred_element_type=jnp.float32)
  p = par_ref[0]
  xe = xt[0:_DIM, :]
  xo = xt[_DIM:2 * _DIM, :]
  xsel = xe + p * (xo - xe)
  pe = jnp.reshape(pe_ref[...], (_DIM, 1))
  y = xsel + pe
  sq = y * y
  colsum = jnp.sum(sq, axis=0, keepdims=True)
  y0 = y[0:1, :]
  inv = lax.rsqrt(jnp.maximum(2.0 * y0 * y0 - colsum, 1e-7))
  yn = y * inv
  h = lax.dot_general(w_ref[...], yn, (((1,), (0,)), ((), ())),
                      preferred_element_type=jnp.float32) + b_ref[...]
  time = jax.nn.sigmoid(h[0:1, :]) * sc_ref[0, 0] + 1.1
  ssq = jnp.sum(h * h, axis=0, keepdims=True) - h[0:1, :] * h[0:1, :]
  s = (time * time - 1.0 / _C) / jnp.maximum(ssq, 1e-8)
  scaled = h * jnp.sqrt(s)
  rowid = lax.broadcasted_iota(jnp.int32, (_DIM, _BBLK), 0)
  outv = jnp.where(rowid == 0, time, scaled)
  out_ref[...] = jnp.reshape(outv[0:63, :], (1, 63, _BBLK))


def _tc_transform(gathered, parity, pe_t, w_pad, b_col, scalars, interpret=False):
  return pl.pallas_call(
      _tc_body,
      grid=(_SEQ, _NB),
      in_specs=[
          pl.BlockSpec((_BBLK, 2 * _DIM), lambda i, j: (i * _NB + j, 0)),
          pl.BlockSpec((1, 1, _BBLK), lambda i, j: (i, 0, j)),
          pl.BlockSpec((1, _DIM, 1), lambda i, j: (i, 0, 0)),
          pl.BlockSpec((_DIM, _DIM), lambda i, j: (0, 0)),
          pl.BlockSpec((_DIM, 1), lambda i, j: (0, 0)),
          pl.BlockSpec((1, 1), lambda i, j: (0, 0)),
      ],
      out_specs=pl.BlockSpec((1, 63, _BBLK), lambda i, j: (i, 0, j)),
      out_shape=jax.ShapeDtypeStruct((_SEQ, 63, _BATCH), jnp.float32),
      interpret=interpret,
  )(gathered, parity, pe_t, w_pad, b_col, scalars)


def kernel(source, embedding, pos_enc, add_scale, W, b, point_scale):
  idx = jnp.transpose(source).reshape(-1).astype(jnp.int32)
  idx3 = (idx // 2).reshape(_NW, _NCHUNK, _CHUNK)
  parity = (idx % 2).astype(jnp.float32).reshape(_SEQ, 1, _BATCH)
  table = embedding.reshape(_VROWS, 2 * _DIM)

  gathered = _sc_gather(idx3, table)

  pe_t = (add_scale * pos_enc[:_SEQ, 0, :]).astype(jnp.float32)[:, :, None]
  w_pad = jnp.zeros((_DIM, _DIM), jnp.float32).at[:63, :].set(W)
  b_col = jnp.zeros((_DIM, 1), jnp.float32).at[:63, 0].set(b)
  scalars = jnp.exp(point_scale).reshape(1, 1)

  out = _tc_transform(gathered, parity, pe_t, w_pad, b_col, scalars)
  return jnp.transpose(out, (2, 0, 1))


# speedup vs baseline: 2.2261x; 1.5660x over previous
import functools
import math

import jax
import jax.numpy as jnp
from jax import lax
from jax.experimental import pallas as pl
from jax.experimental.pallas import tpu as pltpu
from jax.experimental.pallas import tpu_sc as plsc

_C = 1.0
_VOCAB = 1000000
_DIM = 64
_BATCH = 4096
_SEQ = 50

_NC = 2
_NS = 16
_NW = _NC * _NS

_TOKENS = _BATCH * _SEQ
_PER_W = _TOKENS // _NW
_CHUNK = 128
_NCHUNK = _PER_W // _CHUNK
_NSLOT = 5

_BBLK = 2048
_NB = _BATCH // _BBLK


_RB = 2048
_DGRID = (_VOCAB + 2 * _RB - 1) // (2 * _RB)
_VROWS = _DGRID * _RB


def _detile_body(x1_ref, x2_ref, out_ref):
  c = lax.broadcasted_iota(jnp.int32, (_DIM, 2 * _DIM), 1)
  r = lax.broadcasted_iota(jnp.int32, (_DIM, 2 * _DIM), 0)
  p1 = (c == r).astype(jnp.float32)
  p2 = (c == r + _DIM).astype(jnp.float32)
  x1 = x1_ref[...]
  x2 = x2_ref[...]
  out_ref[...] = (
      lax.dot_general(x1, p1, (((0,), (0,)), ((), ())),
                      preferred_element_type=jnp.float32) +
      lax.dot_general(x2, p2, (((0,), (0,)), ((), ())),
                      preferred_element_type=jnp.float32))


def _tc_detile(embT):
  return pl.pallas_call(
      _detile_body,
      grid=(_DGRID,),
      in_specs=[
          pl.BlockSpec((_DIM, _RB), lambda i: (0, 2 * i)),
          pl.BlockSpec((_DIM, _RB),
                       lambda i: (0, jnp.minimum(2 * i + 1, 2 * _DGRID - 2))),
      ],
      out_specs=pl.BlockSpec((_RB, 2 * _DIM), lambda i: (i, 0)),
      out_shape=jax.ShapeDtypeStruct((_VROWS, 2 * _DIM), jnp.float32),
  )(embT, embT)


def _sc_gather(idx3, table):
  mesh = plsc.VectorSubcoreMesh(
      core_axis_name="c", subcore_axis_name="s",
      num_cores=_NC, num_subcores=_NS)

  @functools.partial(
      pl.kernel,
      mesh=mesh,
      compiler_params=pltpu.CompilerParams(use_tc_tiling_on_sc=False),
      out_type=jax.ShapeDtypeStruct((_TOKENS, 2 * _DIM), jnp.float32),
      scratch_types=[
          pltpu.VMEM((_NCHUNK, _CHUNK), jnp.int32),
          pltpu.VMEM((_NSLOT, _CHUNK, 2 * _DIM), jnp.float32),
          pltpu.SemaphoreType.DMA((_NSLOT,)),
          pltpu.SemaphoreType.DMA((_NSLOT,)),
      ],
  )
  def k(idx_hbm, table_hbm, out_hbm, idx_v, rows_v, gsem, wsem):
    wid = lax.axis_index("s") * _NC + lax.axis_index("c")
    base = wid * _PER_W
    pltpu.sync_copy(idx_hbm.at[wid], idx_v)

    def gather(j, s):
      pltpu.make_async_copy(
          table_hbm.at[idx_v.at[j]], rows_v.at[s], gsem.at[s]).start()

    def writeback(j, s):
      return pltpu.make_async_copy(
          rows_v.at[s], out_hbm.at[pl.ds(base + j * _CHUNK, _CHUNK)],
          wsem.at[s])

    for s in range(_NSLOT):
      gather(s, s)

    def body(jj, _):
      for s in range(_NSLOT):
        j = jj * _NSLOT + s
        pltpu.make_async_copy(
            table_hbm.at[idx_v.at[j]], rows_v.at[s], gsem.at[s]).wait()
        writeback(j, s).start()

        @pl.when(jj < _NCHUNK // _NSLOT - 1)
        def _():
          writeback(j, s).wait()
          gather(j + _NSLOT, s)

      return 0

    lax.fori_loop(0, _NCHUNK // _NSLOT, body, 0)

    for s in range(_NSLOT):
      writeback(_NCHUNK - _NSLOT + s, s).wait()

  return k(idx3, table)


def _tc_body(x_ref, par_ref, pe_ref, w_ref, b_ref, sc_ref, out_ref):
  x = x_ref[...]
  ident = (lax.broadcasted_iota(jnp.int32, (128, 128), 0) ==
           lax.broadcasted_iota(jnp.int32, (128, 128), 1)).astype(jnp.float32)
  xt = lax.dot_general(ident, x, (((1,), (1,)), ((), ())),
                       preferred_element_type=jnp.float32)
  p = par_ref[0]
  xe = xt[0:_DIM, :]
  xo = xt[_DIM:2 * _DIM, :]
  xsel = xe + p * (xo - xe)
  pe = jnp.reshape(pe_ref[...], (_DIM, 1))
  y = xsel + pe
  sq = y * y
  colsum = jnp.sum(sq, axis=0, keepdims=True)
  y0 = y[0:1, :]
  inv = lax.rsqrt(jnp.maximum(2.0 * y0 * y0 - colsum, 1e-7))
  yn = y * inv
  h = lax.dot_general(w_ref[...], yn, (((1,), (0,)), ((), ())),
                      preferred_element_type=jnp.float32) + b_ref[...]
  time = jax.nn.sigmoid(h[0:1, :]) * sc_ref[0, 0] + 1.1
  ssq = jnp.sum(h * h, axis=0, keepdims=True) - h[0:1, :] * h[0:1, :]
  s = (time * time - 1.0 / _C) / jnp.maximum(ssq, 1e-8)
  scaled = h * jnp.sqrt(s)
  rowid = lax.broadcasted_iota(jnp.int32, (_DIM, _BBLK), 0)
  outv = jnp.where(rowid == 0, time, scaled)
  out_ref[...] = jnp.reshape(outv[0:63, :], (1, 63, _BBLK))


def _tc_transform(gathered, parity, pe_t, w_pad, b_col, scalars, interpret=False):
  return pl.pallas_call(
      _tc_body,
      grid=(_SEQ, _NB),
      in_specs=[
          pl.BlockSpec((_BBLK, 2 * _DIM), lambda i, j: (i * _NB + j, 0)),
          pl.BlockSpec((1, 1, _BBLK), lambda i, j: (i, 0, j)),
          pl.BlockSpec((1, _DIM, 1), lambda i, j: (i, 0, 0)),
          pl.BlockSpec((_DIM, _DIM), lambda i, j: (0, 0)),
          pl.BlockSpec((_DIM, 1), lambda i, j: (0, 0)),
          pl.BlockSpec((1, 1), lambda i, j: (0, 0)),
      ],
      out_specs=pl.BlockSpec((1, 63, _BBLK), lambda i, j: (i, 0, j)),
      out_shape=jax.ShapeDtypeStruct((_SEQ, 63, _BATCH), jnp.float32),
      interpret=interpret,
  )(gathered, parity, pe_t, w_pad, b_col, scalars)


def kernel(source, embedding, pos_enc, add_scale, W, b, point_scale):
  idx = jnp.transpose(source).reshape(-1).astype(jnp.int32)
  g = idx // (2 * _RB)
  k = idx % (2 * _RB)
  idx3 = (g * _RB + k % _RB).reshape(_NW, _NCHUNK, _CHUNK)
  parity = (k // _RB).astype(jnp.float32).reshape(_SEQ, 1, _BATCH)

  table = _tc_detile(jnp.transpose(embedding))
  gathered = _sc_gather(idx3, table)

  pe_t = (add_scale * pos_enc[:_SEQ, 0, :]).astype(jnp.float32)[:, :, None]
  w_pad = jnp.zeros((_DIM, _DIM), jnp.float32).at[:63, :].set(W)
  b_col = jnp.zeros((_DIM, 1), jnp.float32).at[:63, 0].set(b)
  scalars = jnp.exp(point_scale).reshape(1, 1)

  out = _tc_transform(gathered, parity, pe_t, w_pad, b_col, scalars)
  return jnp.transpose(out, (2, 0, 1))


# speedup vs baseline: 2.2858x; 1.0269x over previous
import functools
import math

import jax
import jax.numpy as jnp
from jax import lax
from jax.experimental import pallas as pl
from jax.experimental.pallas import tpu as pltpu
from jax.experimental.pallas import tpu_sc as plsc

_C = 1.0
_VOCAB = 1000000
_DIM = 64
_BATCH = 4096
_SEQ = 50

_NC = 2
_NS = 16
_NW = _NC * _NS

_TOKENS = _BATCH * _SEQ
_PER_W = _TOKENS // _NW
_CHUNK = 128
_NCHUNK = _PER_W // _CHUNK
_NSLOT = 5

_BBLK = 4096
_NB = _BATCH // _BBLK


_RB = 2048
_DGRID = (_VOCAB + 2 * _RB - 1) // (2 * _RB)
_VROWS = _DGRID * _RB


def _detile_body(x1_ref, x2_ref, out_ref):
  t1 = jnp.transpose(x1_ref[...])
  t2 = jnp.transpose(x2_ref[...])
  out_ref[...] = jnp.concatenate([t1, t2], axis=1)


def _tc_detile(embT):
  return pl.pallas_call(
      _detile_body,
      grid=(_DGRID,),
      in_specs=[
          pl.BlockSpec((_DIM, _RB), lambda i: (0, 2 * i)),
          pl.BlockSpec((_DIM, _RB),
                       lambda i: (0, jnp.minimum(2 * i + 1, 2 * _DGRID - 2))),
      ],
      out_specs=pl.BlockSpec((_RB, 2 * _DIM), lambda i: (i, 0)),
      out_shape=jax.ShapeDtypeStruct((_VROWS, 2 * _DIM), jnp.float32),
  )(embT, embT)


def _sc_gather(idx3, table):
  mesh = plsc.VectorSubcoreMesh(
      core_axis_name="c", subcore_axis_name="s",
      num_cores=_NC, num_subcores=_NS)

  @functools.partial(
      pl.kernel,
      mesh=mesh,
      compiler_params=pltpu.CompilerParams(use_tc_tiling_on_sc=False),
      out_type=jax.ShapeDtypeStruct((_TOKENS, 2 * _DIM), jnp.float32),
      scratch_types=[
          pltpu.VMEM((_NCHUNK, _CHUNK), jnp.int32),
          pltpu.VMEM((_NSLOT, _CHUNK, 2 * _DIM), jnp.float32),
          pltpu.SemaphoreType.DMA((_NSLOT,)),
          pltpu.SemaphoreType.DMA((_NSLOT,)),
      ],
  )
  def k(idx_hbm, table_hbm, out_hbm, idx_v, rows_v, gsem, wsem):
    wid = lax.axis_index("s") * _NC + lax.axis_index("c")
    base = wid * _PER_W
    pltpu.sync_copy(idx_hbm.at[wid], idx_v)

    def gather(j, s):
      pltpu.make_async_copy(
          table_hbm.at[idx_v.at[j]], rows_v.at[s], gsem.at[s]).start()

    def writeback(j, s):
      return pltpu.make_async_copy(
          rows_v.at[s], out_hbm.at[pl.ds(base + j * _CHUNK, _CHUNK)],
          wsem.at[s])

    for s in range(_NSLOT):
      gather(s, s)

    def body(jj, _):
      for s in range(_NSLOT):
        j = jj * _NSLOT + s
        pltpu.make_async_copy(
            table_hbm.at[idx_v.at[j]], rows_v.at[s], gsem.at[s]).wait()
        writeback(j, s).start()

        @pl.when(jj < _NCHUNK // _NSLOT - 1)
        def _():
          writeback(j, s).wait()
          gather(j + _NSLOT, s)

      return 0

    lax.fori_loop(0, _NCHUNK // _NSLOT, body, 0)

    for s in range(_NSLOT):
      writeback(_NCHUNK - _NSLOT + s, s).wait()

  return k(idx3, table)


def _tc_body(x_ref, par_ref, pe_ref, w_ref, b_ref, sc_ref, out_ref):
  x = x_ref[...]
  ident = (lax.broadcasted_iota(jnp.int32, (128, 128), 0) ==
           lax.broadcasted_iota(jnp.int32, (128, 128), 1)).astype(jnp.float32)
  xt = lax.dot_general(ident, x, (((1,), (1,)), ((), ())),
                       preferred_element_type=jnp.float32)
  p = par_ref[0]
  xe = xt[0:_DIM, :]
  xo = xt[_DIM:2 * _DIM, :]
  xsel = xe + p * (xo - xe)
  pe = jnp.reshape(pe_ref[...], (_DIM, 1))
  y = xsel + pe
  sq = y * y
  colsum = jnp.sum(sq, axis=0, keepdims=True)
  y0 = y[0:1, :]
  inv = lax.rsqrt(jnp.maximum(2.0 * y0 * y0 - colsum, 1e-7))
  yn = y * inv
  h = lax.dot_general(w_ref[...], yn, (((1,), (0,)), ((), ())),
                      preferred_element_type=jnp.float32) + b_ref[...]
  time = jax.nn.sigmoid(h[0:1, :]) * sc_ref[0, 0] + 1.1
  ssq = jnp.sum(h * h, axis=0, keepdims=True) - h[0:1, :] * h[0:1, :]
  s = (time * time - 1.0 / _C) / jnp.maximum(ssq, 1e-8)
  scaled = h * jnp.sqrt(s)
  rowid = lax.broadcasted_iota(jnp.int32, (_DIM, _BBLK), 0)
  outv = jnp.where(rowid == 0, time, scaled)
  out_ref[...] = jnp.reshape(outv[0:63, :], (1, 63, _BBLK))


def _tc_transform(gathered, parity, pe_t, w_pad, b_col, scalars, interpret=False):
  return pl.pallas_call(
      _tc_body,
      grid=(_SEQ, _NB),
      in_specs=[
          pl.BlockSpec((_BBLK, 2 * _DIM), lambda i, j: (i * _NB + j, 0)),
          pl.BlockSpec((1, 1, _BBLK), lambda i, j: (i, 0, j)),
          pl.BlockSpec((1, _DIM, 1), lambda i, j: (i, 0, 0)),
          pl.BlockSpec((_DIM, _DIM), lambda i, j: (0, 0)),
          pl.BlockSpec((_DIM, 1), lambda i, j: (0, 0)),
          pl.BlockSpec((1, 1), lambda i, j: (0, 0)),
      ],
      out_specs=pl.BlockSpec((1, 63, _BBLK), lambda i, j: (i, 0, j)),
      out_shape=jax.ShapeDtypeStruct((_SEQ, 63, _BATCH), jnp.float32),
      interpret=interpret,
  )(gathered, parity, pe_t, w_pad, b_col, scalars)


def kernel(source, embedding, pos_enc, add_scale, W, b, point_scale):
  idx = jnp.transpose(source).reshape(-1).astype(jnp.int32)
  g = idx // (2 * _RB)
  k = idx % (2 * _RB)
  idx3 = (g * _RB + k % _RB).reshape(_NW, _NCHUNK, _CHUNK)
  parity = (k // _RB).astype(jnp.float32).reshape(_SEQ, 1, _BATCH)

  table = _tc_detile(jnp.transpose(embedding))
  gathered = _sc_gather(idx3, table)

  pe_t = (add_scale * pos_enc[:_SEQ, 0, :]).astype(jnp.float32)[:, :, None]
  w_pad = jnp.zeros((_DIM, _DIM), jnp.float32).at[:63, :].set(W)
  b_col = jnp.zeros((_DIM, 1), jnp.float32).at[:63, 0].set(b)
  scalars = jnp.exp(point_scale).reshape(1, 1)

  out = _tc_transform(gathered, parity, pe_t, w_pad, b_col, scalars)
  return jnp.transpose(out, (2, 0, 1))
